# baseline (speedup 1.0000x reference)
import functools

import jax
import jax.numpy as jnp
from jax import lax
from jax.experimental import pallas as pl
from jax.experimental.pallas import tpu as pltpu
from jax.experimental.pallas import tpu_sc as plsc

_N1 = 10242
_N2 = 4 * _N1 - 6
_NE = 2 * (3 * _N1 - 6)
_F = 128

_NW = 32

_SPAIR = 2688
_N2P = 16 * _SPAIR
_SZ0 = 1680
_SZ1 = _SPAIR - _SZ0
_SB = 48
_ZROW = _N2P - 1

_ECHUNK = _NE // _NW
_ERB = 128
_ENB = _ECHUNK // _ERB

_MB = 512
_N1P = 10752

_BF = jnp.bfloat16


def _mm_bias_body(x_ref, w_ref, b_ref, o_ref):
    y = jnp.dot(x_ref[...].astype(_BF), w_ref[...],
                preferred_element_type=jnp.float32)
    o_ref[...] = y + b_ref[...]


def _mm_bias_kmaj_body(x_ref, w_ref, b_ref, o_ref):
    y = jnp.dot(x_ref[...].astype(_BF), w_ref[...],
                preferred_element_type=jnp.float32)
    y = y + b_ref[...]
    for k in range(7):
        o_ref[k] = y[:, k * _F : (k + 1) * _F]


def _p1_body(u_ref, x2_ref, w_ref, o_ref):
    xc = jnp.concatenate(
        [u_ref[...].astype(_BF), x2_ref[...].astype(_BF)], axis=1
    )
    y = jnp.dot(xc, w_ref[...], preferred_element_type=jnp.float32)
    for k in range(7):
        o_ref[k] = y[:, k * _F : (k + 1) * _F]


def _scale_shift(ps, gb):
    tot = jnp.sum(ps, axis=0)
    inv_n = 1.0 / _N2
    mu = tot[0:1, :] * inv_n
    var = tot[1:2, :] * inv_n - mu * mu
    scale = gb[0:1, :] * lax.rsqrt(var + 1e-5)
    shift = gb[1:2, :] - mu * scale
    return scale, shift


def _p2_body(h_ref, ps_ref, gb_ref, w_ref, o_ref, ss_ref):
    i = pl.program_id(0)

    @pl.when(i == 0)
    def _init():
        scale, shift = _scale_shift(ps_ref[...], gb_ref[...])
        ss_ref[0:1, :] = scale
        ss_ref[1:2, :] = shift

    y = h_ref[...] * ss_ref[0:1, :] + ss_ref[1:2, :]
    y = jnp.where(y >= 0, y, 0.2 * y)
    rows = lax.broadcasted_iota(jnp.int32, (_MB, 1), 0) + i * _MB
    y = jnp.where(rows < _N2, y, 0.0).astype(_BF)
    z = jnp.dot(y, w_ref[...], preferred_element_type=jnp.float32)
    for k in range(7):
        o_ref[k] = z[:, k * _F : (k + 1) * _F]


def _bnlrelu_body(h_ref, ps_ref, gb_ref, o_ref, ss_ref):
    i = pl.program_id(0)

    @pl.when(i == 0)
    def _init():
        scale, shift = _scale_shift(ps_ref[...], gb_ref[...])
        ss_ref[0:1, :] = scale
        ss_ref[1:2, :] = shift

    y = h_ref[...] * ss_ref[0:1, :] + ss_ref[1:2, :]
    o_ref[...] = jnp.where(y >= 0, y, 0.2 * y)


@functools.lru_cache(maxsize=1)
def _sc_fns():
    mesh = plsc.VectorSubcoreMesh(core_axis_name="c", subcore_axis_name="s")

    @functools.partial(
        pl.kernel,
        out_type=jax.ShapeDtypeStruct((_NE // 2, _F), jnp.float32),
        mesh=mesh,
        scratch_types=[
            pltpu.VMEM((_ECHUNK,), jnp.int32),
            pltpu.VMEM((_ERB, _F), jnp.float32),
            pltpu.VMEM((_ERB // 2, _F), jnp.float32),
            pltpu.SemaphoreType.DMA,
        ],
    )
    def edge_gather(pm_hbm, idx_hbm, out_hbm, idx_v, buf_v, st_v, sem):
        wid = lax.axis_index("s") * 2 + lax.axis_index("c")
        pltpu.sync_copy(idx_hbm.at[wid], idx_v)

        def body(j, carry):
            pltpu.async_copy(
                pm_hbm.at[idx_v.at[pl.ds(j * _ERB, _ERB)]], buf_v, sem
            ).wait()

            def pbody(p, pc):
                for c in range(4):
                    sl = pl.ds(c * 16, 16)
                    st_v[p, sl] = buf_v[2 * p, sl]
                    st_v[p, pl.ds(64 + c * 16, 16)] = buf_v[2 * p + 1, sl]
                return pc

            lax.fori_loop(0, _ERB // 2, pbody, 0)
            pltpu.sync_copy(
                st_v,
                out_hbm.at[pl.ds(wid * (_ECHUNK // 2) + j * (_ERB // 2), _ERB // 2)],
            )
            return carry

        lax.fori_loop(0, _ENB, body, 0)

    @functools.partial(
        pl.kernel,
        out_type=(
            jax.ShapeDtypeStruct((_N2P, _F), jnp.float32),
            jax.ShapeDtypeStruct((_NW, 2, _F), jnp.float32),
        ),
        mesh=mesh,
        scratch_types=[
            pltpu.VMEM((7 * max(_SZ0, _SZ1),), jnp.int32),
            pltpu.VMEM((2, 7 * _SB, _F), jnp.float32),
            pltpu.VMEM((_SB, _F), jnp.float32),
            pltpu.VMEM((2, _F), jnp.float32),
            pltpu.SemaphoreType.DMA,
            pltpu.SemaphoreType.DMA,
        ],
    )
    def gather_acc(tab_hbm, idx_hbm, out_hbm, st_hbm,
                   idx_v, buf_v, acc_v, sums_v, sem0, sem1):
        s = lax.axis_index("s")
        c = lax.axis_index("c")
        wid = s * 2 + c

        def worker(base, sz, nb):
            pltpu.sync_copy(
                idx_hbm.at[pl.ds(base * 7, sz * 7)], idx_v.at[pl.ds(0, sz * 7)]
            )
            for c8 in range(_F // 16):
                sl = pl.ds(c8 * 16, 16)
                sums_v[0, sl] = jnp.zeros((16,), jnp.float32)
                sums_v[1, sl] = jnp.zeros((16,), jnp.float32)

            def fire(j, slot, sem):
                for m in range(3):
                    pltpu.async_copy(
                        tab_hbm.at[
                            idx_v.at[pl.ds(j * (7 * _SB) + m * 112, 112)]
                        ],
                        buf_v.at[slot, pl.ds(m * 112, 112)],
                        sem,
                    )

            def drain(slot, sem):
                for m in range(3):
                    pltpu.make_async_copy(
                        tab_hbm.at[idx_v.at[pl.ds(m * 112, 112)]],
                        buf_v.at[slot, pl.ds(m * 112, 112)],
                        sem,
                    ).wait()

            def acc_store(j, slot):
                def rbody(r, rc):
                    for c8 in range(_F // 16):
                        sl = pl.ds(c8 * 16, 16)
                        v = buf_v[slot, 7 * r, sl]
                        for k in range(1, 7):
                            v = v + buf_v[slot, 7 * r + k, sl]
                        acc_v[r, sl] = v
                        sums_v[0, sl] += v
                        sums_v[1, sl] += v * v
                    return rc

                lax.fori_loop(0, _SB, rbody, 0)
                pltpu.sync_copy(
                    acc_v, out_hbm.at[pl.ds(base + j * _SB, _SB)]
                )

            fire(0, 0, sem0)

            def body(pair, carry):
                j0 = 2 * pair
                fire(j0 + 1, 1, sem1)
                drain(0, sem0)
                acc_store(j0, 0)

                if nb % 2 == 1:
                    fire(j0 + 2, 0, sem0)
                else:
                    @pl.when(pair < nb // 2 - 1)
                    def _prefetch():
                        fire(j0 + 2, 0, sem0)

                drain(1, sem1)
                acc_store(j0 + 1, 1)
                return carry

            lax.fori_loop(0, nb // 2, body, 0)
            if nb % 2 == 1:
                drain(0, sem0)
                acc_store(nb - 1, 0)
            pltpu.sync_copy(sums_v, st_hbm.at[wid])

        @pl.when(c == 0)
        def _core0():
            worker(s * _SPAIR, _SZ0, _SZ0 // _SB)

        @pl.when(c == 1)
        def _core1():
            worker(s * _SPAIR + _SZ0, _SZ1, _SZ1 // _SB)

    return edge_gather, gather_acc


def _mm_bias_call(xp, w_t, b, nrows):
    grid = nrows // _MB
    return pl.pallas_call(
        _mm_bias_body,
        grid=(grid,),
        in_specs=[
            pl.BlockSpec((_MB, 2 * _F), lambda i: (i, 0)),
            pl.BlockSpec((2 * _F, 7 * _F), lambda i: (0, 0)),
            pl.BlockSpec((1, 7 * _F), lambda i: (0, 0)),
        ],
        out_specs=pl.BlockSpec((_MB, 7 * _F), lambda i: (i, 0)),
        out_shape=jax.ShapeDtypeStruct((nrows, 7 * _F), jnp.float32),
    )(xp, w_t, b)


def _mm_bias_kmaj_call(xp, w_t, b):
    grid = _N1P // _MB
    return pl.pallas_call(
        _mm_bias_kmaj_body,
        grid=(grid,),
        in_specs=[
            pl.BlockSpec((_MB, 2 * _F), lambda i: (i, 0)),
            pl.BlockSpec((2 * _F, 7 * _F), lambda i: (0, 0)),
            pl.BlockSpec((1, 7 * _F), lambda i: (0, 0)),
        ],
        out_specs=pl.BlockSpec((7, _MB, _F), lambda i: (0, i, 0)),
        out_shape=jax.ShapeDtypeStruct((7, _N1P, _F), jnp.float32),
    )(xp, w_t, b)


def _p1_call(up_p, x2p, w1p):
    grid = _N2P // _MB
    return pl.pallas_call(
        _p1_body,
        grid=(grid,),
        in_specs=[
            pl.BlockSpec((_MB, _F), lambda i: (i, 0)),
            pl.BlockSpec((_MB, _F), lambda i: (i, 0)),
            pl.BlockSpec((2 * _F, 7 * _F), lambda i: (0, 0)),
        ],
        out_specs=pl.BlockSpec((7, _MB, _F), lambda i: (0, i, 0)),
        out_shape=jax.ShapeDtypeStruct((7, _N2P, _F), jnp.float32),
    )(up_p, x2p, w1p)


def _p2_call(h, ps, gb, w2p):
    grid = _N2P // _MB
    return pl.pallas_call(
        _p2_body,
        grid=(grid,),
        in_specs=[
            pl.BlockSpec((_MB, _F), lambda i: (i, 0)),
            pl.BlockSpec((_NW, 2, _F), lambda i: (0, 0, 0)),
            pl.BlockSpec((2, _F), lambda i: (0, 0)),
            pl.BlockSpec((_F, 7 * _F), lambda i: (0, 0)),
        ],
        out_specs=pl.BlockSpec((7, _MB, _F), lambda i: (0, i, 0)),
        out_shape=jax.ShapeDtypeStruct((7, _N2P, _F), jnp.float32),
        scratch_shapes=[pltpu.VMEM((8, _F), jnp.float32)],
    )(h, ps, gb, w2p)


def _bnlrelu_call(h, ps, gb):
    grid = (_N2 + _MB - 1) // _MB
    return pl.pallas_call(
        _bnlrelu_body,
        grid=(grid,),
        in_specs=[
            pl.BlockSpec((_MB, _F), lambda i: (i, 0)),
            pl.BlockSpec((_NW, 2, _F), lambda i: (0, 0, 0)),
            pl.BlockSpec((2, _F), lambda i: (0, 0)),
        ],
        out_specs=pl.BlockSpec((_MB, _F), lambda i: (i, 0)),
        out_shape=jax.ShapeDtypeStruct((_N2, _F), jnp.float32),
        scratch_shapes=[pltpu.VMEM((8, _F), jnp.float32)],
    )(h, ps, gb)


def kernel(x1, x2, upconv_center_indices, upconv_edge_indices, neigh_orders,
           W_up, b_up, W1, b1, gamma1, beta1, W2, b2, gamma2, beta2):
    f32 = jnp.float32

    w_pm = W_up.reshape(7, 64, 2, 2 * _F).mean(axis=2)
    w_pm = jnp.concatenate([w_pm, jnp.zeros_like(w_pm)], axis=1)
    w_up_t = W_up.T.astype(_BF)
    w_pm_t = w_pm.reshape(7 * _F, 2 * _F).T.astype(_BF)
    w1p = W1.reshape(_F, 7, 2 * _F).transpose(2, 1, 0).reshape(2 * _F, 7 * _F)
    w2p = W2.reshape(_F, 7, _F).transpose(2, 1, 0).reshape(_F, 7 * _F)
    w1p = w1p.astype(_BF)
    w2p = w2p.astype(_BF)
    gb1 = jnp.stack([gamma1, beta1]).astype(f32)
    gb2 = jnp.stack([gamma2, beta2]).astype(f32)
    b_pm = b_up.reshape(7, 64, 2).mean(axis=2)
    b_pm = jnp.concatenate([b_pm, jnp.zeros_like(b_pm)], axis=1)
    b_pm = b_pm.reshape(7 * _F)[None, :]

    edge_gather, gather_acc = _sc_fns()

    x1p = jnp.pad(x1, ((0, _N1P - _N1), (0, 0)))
    pm_p = _mm_bias_kmaj_call(x1p, w_pm_t, b_pm)
    pm_flat = pm_p.reshape(7 * _N1P, _F)
    e = upconv_edge_indices
    eidx = ((e % 7) * _N1P + e // 7).reshape(_NW, _ECHUNK)
    u2m = edge_gather(pm_flat, eidx)
    u1_p = _mm_bias_call(x1p[:1536], w_up_t, b_up[None, :], 1536)
    u1 = u1_p.reshape(1536 * 7, _F)[:_N1]

    up_out = jnp.concatenate([u1, u2m], axis=0)
    up_out_p = jnp.pad(up_out, ((0, _N2P - _N2), (0, 0)))
    x2p = jnp.pad(x2, ((0, _N2P - _N2), (0, 0)))

    ng2 = (neigh_orders.reshape(_N2, 7)
           + jnp.arange(7, dtype=jnp.int32)[None, :] * _N2P)
    nidx = jnp.pad(ng2.reshape(-1), (0, (_N2P - _N2) * 7),
                   constant_values=_ZROW)

    p1 = _p1_call(up_out_p, x2p, w1p)
    h1, ps1 = gather_acc(p1.reshape(7 * _N2P, _F), nidx)

    p2 = _p2_call(h1, ps1, gb1, w2p)
    h2, ps2 = gather_acc(p2.reshape(7 * _N2P, _F), nidx)

    return _bnlrelu_call(h2, ps2, gb2)


# speedup vs baseline: 1.0159x; 1.0159x over previous
import functools

import jax
import jax.numpy as jnp
from jax import lax
from jax.experimental import pallas as pl
from jax.experimental.pallas import tpu as pltpu
from jax.experimental.pallas import tpu_sc as plsc

_N1 = 10242
_N2 = 4 * _N1 - 6
_NE = 2 * (3 * _N1 - 6)
_F = 128

_NW = 32

_SPAIR = 2688
_N2P = 16 * _SPAIR
_SZ0 = 1632
_SZ1 = _SPAIR - _SZ0
_SB = 48
_ZROW = _N2P - 1

_ECHUNK = _NE // _NW
_ERB = 128
_ENB = _ECHUNK // _ERB

_MB = 512
_N1P = 10752

_BF = jnp.bfloat16


def _mm_bias_body(x_ref, w_ref, b_ref, o_ref):
    y = jnp.dot(x_ref[...].astype(_BF), w_ref[...],
                preferred_element_type=jnp.float32)
    o_ref[...] = y + b_ref[...]


def _mm_bias_kmaj_body(x_ref, w_ref, b_ref, o_ref):
    y = jnp.dot(x_ref[...].astype(_BF), w_ref[...],
                preferred_element_type=jnp.float32)
    y = y + b_ref[...]
    for k in range(7):
        o_ref[k] = y[:, k * _F : (k + 1) * _F]


def _p1_body(u_ref, x2_ref, w_ref, o_ref):
    xc = jnp.concatenate(
        [u_ref[...].astype(_BF), x2_ref[...].astype(_BF)], axis=1
    )
    y = jnp.dot(xc, w_ref[...], preferred_element_type=jnp.float32)
    for k in range(7):
        o_ref[k] = y[:, k * _F : (k + 1) * _F]


def _scale_shift(ps, gb):
    tot = jnp.sum(ps, axis=0)
    inv_n = 1.0 / _N2
    mu = tot[0:1, :] * inv_n
    var = tot[1:2, :] * inv_n - mu * mu
    scale = gb[0:1, :] * lax.rsqrt(var + 1e-5)
    shift = gb[1:2, :] - mu * scale
    return scale, shift


def _p2_body(h_ref, ps_ref, gb_ref, w_ref, o_ref, ss_ref):
    i = pl.program_id(0)

    @pl.when(i == 0)
    def _init():
        scale, shift = _scale_shift(ps_ref[...], gb_ref[...])
        ss_ref[0:1, :] = scale
        ss_ref[1:2, :] = shift

    y = h_ref[...] * ss_ref[0:1, :] + ss_ref[1:2, :]
    y = jnp.where(y >= 0, y, 0.2 * y)
    rows = lax.broadcasted_iota(jnp.int32, (_MB, 1), 0) + i * _MB
    y = jnp.where(rows < _N2, y, 0.0).astype(_BF)
    z = jnp.dot(y, w_ref[...], preferred_element_type=jnp.float32)
    for k in range(7):
        o_ref[k] = z[:, k * _F : (k + 1) * _F]


def _bnlrelu_body(h_ref, ps_ref, gb_ref, o_ref, ss_ref):
    i = pl.program_id(0)

    @pl.when(i == 0)
    def _init():
        scale, shift = _scale_shift(ps_ref[...], gb_ref[...])
        ss_ref[0:1, :] = scale
        ss_ref[1:2, :] = shift

    y = h_ref[...] * ss_ref[0:1, :] + ss_ref[1:2, :]
    o_ref[...] = jnp.where(y >= 0, y, 0.2 * y)


@functools.lru_cache(maxsize=1)
def _sc_fns():
    mesh = plsc.VectorSubcoreMesh(core_axis_name="c", subcore_axis_name="s")

    @functools.partial(
        pl.kernel,
        out_type=jax.ShapeDtypeStruct((_NE // 2, _F), jnp.float32),
        mesh=mesh,
        scratch_types=[
            pltpu.VMEM((_ECHUNK,), jnp.int32),
            pltpu.VMEM((_ERB, _F), jnp.float32),
            pltpu.VMEM((_ERB // 2, _F), jnp.float32),
            pltpu.SemaphoreType.DMA,
        ],
    )
    def edge_gather(pm_hbm, idx_hbm, out_hbm, idx_v, buf_v, st_v, sem):
        wid = lax.axis_index("s") * 2 + lax.axis_index("c")
        pltpu.sync_copy(idx_hbm.at[wid], idx_v)

        def body(j, carry):
            pltpu.async_copy(
                pm_hbm.at[idx_v.at[pl.ds(j * _ERB, _ERB)]], buf_v, sem
            ).wait()

            def pbody(p, pc):
                for c in range(4):
                    sl = pl.ds(c * 16, 16)
                    st_v[p, sl] = buf_v[2 * p, sl]
                    st_v[p, pl.ds(64 + c * 16, 16)] = buf_v[2 * p + 1, sl]
                return pc

            lax.fori_loop(0, _ERB // 2, pbody, 0)
            pltpu.sync_copy(
                st_v,
                out_hbm.at[pl.ds(wid * (_ECHUNK // 2) + j * (_ERB // 2), _ERB // 2)],
            )
            return carry

        lax.fori_loop(0, _ENB, body, 0)

    @functools.partial(
        pl.kernel,
        out_type=(
            jax.ShapeDtypeStruct((_N2P, _F), jnp.float32),
            jax.ShapeDtypeStruct((_NW, 2, _F), jnp.float32),
        ),
        mesh=mesh,
        scratch_types=[
            pltpu.VMEM((7 * max(_SZ0, _SZ1),), jnp.int32),
            pltpu.VMEM((2, 7 * _SB, _F), jnp.float32),
            pltpu.VMEM((_SB, _F), jnp.float32),
            pltpu.VMEM((2, _F), jnp.float32),
            pltpu.SemaphoreType.DMA,
            pltpu.SemaphoreType.DMA,
        ],
    )
    def gather_acc(tab_hbm, idx_hbm, out_hbm, st_hbm,
                   idx_v, buf_v, acc_v, sums_v, sem0, sem1):
        s = lax.axis_index("s")
        c = lax.axis_index("c")
        wid = s * 2 + c

        def worker(base, sz, nb):
            pltpu.sync_copy(
                idx_hbm.at[pl.ds(base * 7, sz * 7)], idx_v.at[pl.ds(0, sz * 7)]
            )
            for c8 in range(_F // 16):
                sl = pl.ds(c8 * 16, 16)
                sums_v[0, sl] = jnp.zeros((16,), jnp.float32)
                sums_v[1, sl] = jnp.zeros((16,), jnp.float32)

            def fire(j, slot, sem):
                for m in range(3):
                    pltpu.async_copy(
                        tab_hbm.at[
                            idx_v.at[pl.ds(j * (7 * _SB) + m * 112, 112)]
                        ],
                        buf_v.at[slot, pl.ds(m * 112, 112)],
                        sem,
                    )

            def drain(slot, sem):
                for m in range(3):
                    pltpu.make_async_copy(
                        tab_hbm.at[idx_v.at[pl.ds(m * 112, 112)]],
                        buf_v.at[slot, pl.ds(m * 112, 112)],
                        sem,
                    ).wait()

            def acc_store(j, slot):
                def rbody(r, rc):
                    for c8 in range(_F // 16):
                        sl = pl.ds(c8 * 16, 16)
                        v = buf_v[slot, 7 * r, sl]
                        for k in range(1, 7):
                            v = v + buf_v[slot, 7 * r + k, sl]
                        acc_v[r, sl] = v
                        sums_v[0, sl] += v
                        sums_v[1, sl] += v * v
                    return rc

                lax.fori_loop(0, _SB, rbody, 0)
                pltpu.sync_copy(
                    acc_v, out_hbm.at[pl.ds(base + j * _SB, _SB)]
                )

            fire(0, 0, sem0)

            def body(pair, carry):
                j0 = 2 * pair
                fire(j0 + 1, 1, sem1)
                drain(0, sem0)
                acc_store(j0, 0)

                if nb % 2 == 1:
                    fire(j0 + 2, 0, sem0)
                else:
                    @pl.when(pair < nb // 2 - 1)
                    def _prefetch():
                        fire(j0 + 2, 0, sem0)

                drain(1, sem1)
                acc_store(j0 + 1, 1)
                return carry

            lax.fori_loop(0, nb // 2, body, 0)
            if nb % 2 == 1:
                drain(0, sem0)
                acc_store(nb - 1, 0)
            pltpu.sync_copy(sums_v, st_hbm.at[wid])

        @pl.when(c == 0)
        def _core0():
            worker(s * _SPAIR, _SZ0, _SZ0 // _SB)

        @pl.when(c == 1)
        def _core1():
            worker(s * _SPAIR + _SZ0, _SZ1, _SZ1 // _SB)

    return edge_gather, gather_acc


def _mm_bias_call(xp, w_t, b, nrows):
    grid = nrows // _MB
    return pl.pallas_call(
        _mm_bias_body,
        grid=(grid,),
        in_specs=[
            pl.BlockSpec((_MB, 2 * _F), lambda i: (i, 0)),
            pl.BlockSpec((2 * _F, 7 * _F), lambda i: (0, 0)),
            pl.BlockSpec((1, 7 * _F), lambda i: (0, 0)),
        ],
        out_specs=pl.BlockSpec((_MB, 7 * _F), lambda i: (i, 0)),
        out_shape=jax.ShapeDtypeStruct((nrows, 7 * _F), jnp.float32),
    )(xp, w_t, b)


def _mm_bias_kmaj_call(xp, w_t, b):
    grid = _N1P // _MB
    return pl.pallas_call(
        _mm_bias_kmaj_body,
        grid=(grid,),
        in_specs=[
            pl.BlockSpec((_MB, 2 * _F), lambda i: (i, 0)),
            pl.BlockSpec((2 * _F, 7 * _F), lambda i: (0, 0)),
            pl.BlockSpec((1, 7 * _F), lambda i: (0, 0)),
        ],
        out_specs=pl.BlockSpec((7, _MB, _F), lambda i: (0, i, 0)),
        out_shape=jax.ShapeDtypeStruct((7, _N1P, _F), jnp.float32),
    )(xp, w_t, b)


def _p1_call(up_p, x2p, w1p):
    grid = _N2P // _MB
    return pl.pallas_call(
        _p1_body,
        grid=(grid,),
        in_specs=[
            pl.BlockSpec((_MB, _F), lambda i: (i, 0)),
            pl.BlockSpec((_MB, _F), lambda i: (i, 0)),
            pl.BlockSpec((2 * _F, 7 * _F), lambda i: (0, 0)),
        ],
        out_specs=pl.BlockSpec((7, _MB, _F), lambda i: (0, i, 0)),
        out_shape=jax.ShapeDtypeStruct((7, _N2P, _F), jnp.float32),
    )(up_p, x2p, w1p)


def _p2_call(h, ps, gb, w2p):
    grid = _N2P // _MB
    return pl.pallas_call(
        _p2_body,
        grid=(grid,),
        in_specs=[
            pl.BlockSpec((_MB, _F), lambda i: (i, 0)),
            pl.BlockSpec((_NW, 2, _F), lambda i: (0, 0, 0)),
            pl.BlockSpec((2, _F), lambda i: (0, 0)),
            pl.BlockSpec((_F, 7 * _F), lambda i: (0, 0)),
        ],
        out_specs=pl.BlockSpec((7, _MB, _F), lambda i: (0, i, 0)),
        out_shape=jax.ShapeDtypeStruct((7, _N2P, _F), jnp.float32),
        scratch_shapes=[pltpu.VMEM((8, _F), jnp.float32)],
    )(h, ps, gb, w2p)


def _bnlrelu_call(h, ps, gb):
    grid = (_N2 + _MB - 1) // _MB
    return pl.pallas_call(
        _bnlrelu_body,
        grid=(grid,),
        in_specs=[
            pl.BlockSpec((_MB, _F), lambda i: (i, 0)),
            pl.BlockSpec((_NW, 2, _F), lambda i: (0, 0, 0)),
            pl.BlockSpec((2, _F), lambda i: (0, 0)),
        ],
        out_specs=pl.BlockSpec((_MB, _F), lambda i: (i, 0)),
        out_shape=jax.ShapeDtypeStruct((_N2, _F), jnp.float32),
        scratch_shapes=[pltpu.VMEM((8, _F), jnp.float32)],
    )(h, ps, gb)


def kernel(x1, x2, upconv_center_indices, upconv_edge_indices, neigh_orders,
           W_up, b_up, W1, b1, gamma1, beta1, W2, b2, gamma2, beta2):
    f32 = jnp.float32

    w_pm = W_up.reshape(7, 64, 2, 2 * _F).mean(axis=2)
    w_pm = jnp.concatenate([w_pm, jnp.zeros_like(w_pm)], axis=1)
    w_up_t = W_up.T.astype(_BF)
    w_pm_t = w_pm.reshape(7 * _F, 2 * _F).T.astype(_BF)
    w1p = W1.reshape(_F, 7, 2 * _F).transpose(2, 1, 0).reshape(2 * _F, 7 * _F)
    w2p = W2.reshape(_F, 7, _F).transpose(2, 1, 0).reshape(_F, 7 * _F)
    w1p = w1p.astype(_BF)
    w2p = w2p.astype(_BF)
    gb1 = jnp.stack([gamma1, beta1]).astype(f32)
    gb2 = jnp.stack([gamma2, beta2]).astype(f32)
    b_pm = b_up.reshape(7, 64, 2).mean(axis=2)
    b_pm = jnp.concatenate([b_pm, jnp.zeros_like(b_pm)], axis=1)
    b_pm = b_pm.reshape(7 * _F)[None, :]

    edge_gather, gather_acc = _sc_fns()

    x1p = jnp.pad(x1, ((0, _N1P - _N1), (0, 0)))
    pm_p = _mm_bias_kmaj_call(x1p, w_pm_t, b_pm)
    pm_flat = pm_p.reshape(7 * _N1P, _F)
    e = upconv_edge_indices
    eidx = ((e % 7) * _N1P + e // 7).reshape(_NW, _ECHUNK)
    u2m = edge_gather(pm_flat, eidx)
    u1_p = _mm_bias_call(x1p[:1536], w_up_t, b_up[None, :], 1536)
    u1 = u1_p.reshape(1536 * 7, _F)[:_N1]

    up_out = jnp.concatenate([u1, u2m], axis=0)
    up_out_p = jnp.pad(up_out, ((0, _N2P - _N2), (0, 0)))
    x2p = jnp.pad(x2, ((0, _N2P - _N2), (0, 0)))

    ng2 = (neigh_orders.reshape(_N2, 7)
           + jnp.arange(7, dtype=jnp.int32)[None, :] * _N2P)
    nidx = jnp.pad(ng2.reshape(-1), (0, (_N2P - _N2) * 7),
                   constant_values=_ZROW)

    p1 = _p1_call(up_out_p, x2p, w1p)
    h1, ps1 = gather_acc(p1.reshape(7 * _N2P, _F), nidx)

    p2 = _p2_call(h1, ps1, gb1, w2p)
    h2, ps2 = gather_acc(p2.reshape(7 * _N2P, _F), nidx)

    return _bnlrelu_call(h2, ps2, gb2)
